# Initial kernel scaffold; baseline (speedup 1.0000x reference)
import jax
import jax.numpy as jnp
from jax.experimental import pallas as pl


def kernel(x, edge_index, W1, b1, W2, b2, W3, b3):
    raise NotImplementedError("write your pallas kernel here")


# speedup vs baseline: 15.3063x; 15.3063x over previous
import functools

import jax
import jax.numpy as jnp
from jax import lax
from jax.experimental import pallas as pl
from jax.experimental.pallas import tpu as pltpu
from jax.experimental.pallas import tpu_sc as plsc

N = 10000
E = 320000
N_PAD = 10240
NC, NS = 2, 16
NW = NC * NS
CHUNK = 128
NCHUNK = 80
NBUF = 4
NGROUP = NCHUNK // NBUF
E_PAD = NW * NCHUNK * CHUNK
RPS = N_PAD // NS


def _mesh():
    return plsc.VectorSubcoreMesh(core_axis_name="c", subcore_axis_name="s",
                                  num_cores=NC, num_subcores=NS)


def _make_deg_kernel():
    F = 16

    @functools.partial(
        pl.kernel,
        mesh=_mesh(),
        out_type=jax.ShapeDtypeStruct((NC, N_PAD, F), jnp.float32),
        scratch_types=[
            pltpu.VMEM((NCHUNK, CHUNK), jnp.int32),
            pltpu.VMEM((CHUNK, F), jnp.float32),
            pltpu.VMEM_SHARED((N_PAD, F), jnp.float32),
            pltpu.SemaphoreType.DMA,
        ],
        compiler_params=pltpu.CompilerParams(use_tc_tiling_on_sc=False),
    )
    def deg_kernel(dst_hbm, zeros_hbm, ones_hbm, out_hbm, dst_v, ones_v,
                   acc_sh, sem):
        c = lax.axis_index("c")
        s = lax.axis_index("s")
        wid = s * NC + c
        off = pl.multiple_of(s * RPS, 8)
        pltpu.sync_copy(zeros_hbm, acc_sh.at[pl.ds(off, RPS)])
        pltpu.sync_copy(ones_hbm, ones_v)
        pltpu.sync_copy(dst_hbm.at[wid], dst_v)
        plsc.subcore_barrier()

        def fire(g, _):
            for b in range(NBUF):
                j = g * NBUF + b
                pltpu.async_copy(ones_v, acc_sh.at[dst_v.at[j]], sem,
                                 add=True)
            return ()

        lax.fori_loop(0, NGROUP, fire, (), unroll=False)

        def drain(g, _):
            for b in range(NBUF):
                j = g * NBUF + b
                pltpu.make_async_copy(ones_v, acc_sh.at[dst_v.at[j]],
                                      sem).wait()
            return ()

        lax.fori_loop(0, NGROUP, drain, (), unroll=False)
        plsc.subcore_barrier()
        pltpu.sync_copy(acc_sh.at[pl.ds(off, RPS)], out_hbm.at[c, pl.ds(off, RPS)])

    return deg_kernel


def _make_agg_kernel(F):
    @functools.partial(
        pl.kernel,
        mesh=_mesh(),
        out_type=jax.ShapeDtypeStruct((NC, N_PAD, F), jnp.float32),
        scratch_types=[
            pltpu.VMEM((NCHUNK, CHUNK), jnp.int32),
            pltpu.VMEM((NCHUNK, CHUNK), jnp.int32),
            pltpu.VMEM((NBUF, CHUNK, F), jnp.float32),
            pltpu.VMEM_SHARED((N_PAD, F), jnp.float32),
            [pltpu.SemaphoreType.DMA] * NBUF,
            [pltpu.SemaphoreType.DMA] * NBUF,
        ],
        compiler_params=pltpu.CompilerParams(use_tc_tiling_on_sc=False),
    )
    def agg_kernel(g_hbm, src_hbm, dst_hbm, zeros_hbm, out_hbm, src_v, dst_v,
                   rows_v, acc_sh, gsems, ssems):
        c = lax.axis_index("c")
        s = lax.axis_index("s")
        wid = s * NC + c
        off = pl.multiple_of(s * RPS, 8)
        pltpu.sync_copy(zeros_hbm, acc_sh.at[pl.ds(off, RPS)])
        pltpu.sync_copy(src_hbm.at[wid], src_v)
        pltpu.sync_copy(dst_hbm.at[wid], dst_v)
        plsc.subcore_barrier()

        def gather(j, b):
            pltpu.async_copy(g_hbm.at[src_v.at[j]], rows_v.at[b], gsems[b])

        def gather_wait(j, b):
            pltpu.make_async_copy(g_hbm.at[src_v.at[j]], rows_v.at[b],
                                  gsems[b]).wait()

        def scat(j, b):
            pltpu.async_copy(rows_v.at[b], acc_sh.at[dst_v.at[j]], ssems[b],
                             add=True)

        def scat_wait(j, b):
            pltpu.make_async_copy(rows_v.at[b], acc_sh.at[dst_v.at[j]],
                                  ssems[b]).wait()

        def slot(j, b, do_sw, do_g):
            gather_wait(j, b)
            scat(j, b)
            bn = (b + 2) % NBUF
            if do_sw:
                scat_wait(j - 2, bn)
            if do_g:
                gather(j + 2, bn)

        gather(0, 0)
        gather(1, 1)
        slot(0, 0, False, True)
        slot(1, 1, False, True)
        slot(2, 2, True, True)
        slot(3, 3, True, True)

        def body(g, _):
            for b in range(NBUF):
                slot(g * NBUF + b, b, True, True)
            return ()

        lax.fori_loop(1, NGROUP - 1, body, (), unroll=False)
        j0 = (NGROUP - 1) * NBUF
        slot(j0 + 0, 0, True, True)
        slot(j0 + 1, 1, True, True)
        slot(j0 + 2, 2, False, False)
        slot(j0 + 3, 3, False, False)
        for b in range(NBUF):
            scat_wait(j0 + b, b)

        plsc.subcore_barrier()
        pltpu.sync_copy(acc_sh.at[pl.ds(off, RPS)], out_hbm.at[c, pl.ds(off, RPS)])

    return agg_kernel


@functools.cache
def _sc_kernels():
    return (_make_deg_kernel(), _make_agg_kernel(64), _make_agg_kernel(32),
            _make_agg_kernel(48))


_BR = 512
_TC_GRID = N_PAD // _BR
_BRF = 400
_TC_GRID_F = N // _BRF


def _dinv_of(degp_ref):
    deg = degp_ref[0, :, 0:1] + degp_ref[1, :, 0:1] + 1.0
    return lax.rsqrt(deg)


def _tc1_body(x_ref, w_ref, degp_ref, h_ref, g_ref):
    h = jnp.dot(x_ref[...], w_ref[...], preferred_element_type=jnp.float32)
    dinv = _dinv_of(degp_ref)
    h_ref[...] = h
    g_ref[...] = h * dinv


def _tc_mid_body(h_ref, degp_ref, accp_ref, b_ref, w_ref, hn_ref, gn_ref):
    dinv = _dinv_of(degp_ref)
    acc = accp_ref[0] + accp_ref[1]
    z = jnp.maximum(acc * dinv + h_ref[...] * (dinv * dinv) + b_ref[...], 0.0)
    hn = jnp.dot(z, w_ref[...], preferred_element_type=jnp.float32)
    hn_ref[...] = hn
    gn_ref[...] = hn * dinv


def _tc_fin_body(h_ref, degp_ref, accp_ref, b_ref, out_ref):
    dinv = _dinv_of(degp_ref)
    acc = accp_ref[0] + accp_ref[1]
    z = acc * dinv + h_ref[...] * (dinv * dinv) + b_ref[...]
    z = z[:, :40]
    m = jnp.max(z, axis=1, keepdims=True)
    e = jnp.exp(z - m)
    out_ref[...] = e / jnp.sum(e, axis=1, keepdims=True)


def _row_spec(r, f):
    return pl.BlockSpec((r, f), lambda i: (i, 0))


def _degp_spec(r):
    return pl.BlockSpec((NC, r, 16), lambda i: (0, i, 0))


def _accp_spec(r, f):
    return pl.BlockSpec((NC, r, f), lambda i: (0, i, 0))


def _full_spec(a, b):
    return pl.BlockSpec((a, b), lambda i: (0, 0))


def _tc1(x, w1, degp):
    return pl.pallas_call(
        _tc1_body,
        grid=(_TC_GRID,),
        in_specs=[_row_spec(_BR, 128), _full_spec(128, 64), _degp_spec(_BR)],
        out_specs=[_row_spec(_BR, 64), _row_spec(_BR, 64)],
        out_shape=[
            jax.ShapeDtypeStruct((N_PAD, 64), jnp.float32),
            jax.ShapeDtypeStruct((N_PAD, 64), jnp.float32),
        ],
    )(x, w1, degp)


def _tc_mid(h, degp, accp, b, w, fin, fout):
    return pl.pallas_call(
        _tc_mid_body,
        grid=(_TC_GRID,),
        in_specs=[_row_spec(_BR, fin), _degp_spec(_BR), _accp_spec(_BR, fin),
                  _full_spec(1, fin), _full_spec(fin, fout)],
        out_specs=[_row_spec(_BR, fout), _row_spec(_BR, fout)],
        out_shape=[
            jax.ShapeDtypeStruct((N_PAD, fout), jnp.float32),
            jax.ShapeDtypeStruct((N_PAD, fout), jnp.float32),
        ],
    )(h, degp, accp, b, w)


def _tc_fin(h, degp, accp, b):
    return pl.pallas_call(
        _tc_fin_body,
        grid=(_TC_GRID_F,),
        in_specs=[_row_spec(_BRF, 48), _degp_spec(_BRF),
                  _accp_spec(_BRF, 48), _full_spec(1, 48)],
        out_specs=_row_spec(_BRF, 40),
        out_shape=jax.ShapeDtypeStruct((N, 40), jnp.float32),
    )(h, degp, accp, b)


def kernel(x, edge_index, W1, b1, W2, b2, W3, b3):
    pad_idx = jnp.full((E_PAD - E,), N, jnp.int32)
    src = jnp.concatenate([edge_index[0].astype(jnp.int32), pad_idx])
    dst = jnp.concatenate([edge_index[1].astype(jnp.int32), pad_idx])
    src = src.reshape(NW, NCHUNK, CHUNK)
    dst = dst.reshape(NW, NCHUNK, CHUNK)
    xp = jnp.pad(x, ((0, N_PAD - N), (0, 0)))

    z16 = jnp.zeros((RPS, 16), jnp.float32)
    z64 = jnp.zeros((RPS, 64), jnp.float32)
    z32 = jnp.zeros((RPS, 32), jnp.float32)
    z48 = jnp.zeros((RPS, 48), jnp.float32)
    ones = jnp.ones((CHUNK, 16), jnp.float32)

    W3p = jnp.pad(W3, ((0, 0), (0, 8)))
    b3p = jnp.pad(b3, (0, 8))

    deg_k, agg64, agg32, agg48 = _sc_kernels()
    degp = deg_k(dst, z16, ones)

    h1, g1 = _tc1(xp, W1, degp)
    acc1 = agg64(g1, src, dst, z64)
    h2, g2 = _tc_mid(h1, degp, acc1, b1.reshape(1, 64), W2, 64, 32)
    acc2 = agg32(g2, src, dst, z32)
    h3, g3 = _tc_mid(h2, degp, acc2, b2.reshape(1, 32), W3p, 32, 48)
    acc3 = agg48(g3, src, dst, z48)
    return _tc_fin(h3, degp, acc3, b3p.reshape(1, 48))


# speedup vs baseline: 16.1261x; 1.0536x over previous
import functools

import jax
import jax.numpy as jnp
from jax import lax
from jax.experimental import pallas as pl
from jax.experimental.pallas import tpu as pltpu
from jax.experimental.pallas import tpu_sc as plsc

N = 10000
E = 320000
N_PAD = 10240
NC, NS = 2, 16
CHUNK = 128
NBUF = 4
NCH0 = 120
NCH1 = 40
NCH_MAX = max(NCH0, NCH1)
TOT_CHUNKS = NS * (NCH0 + NCH1)
E_ALLOC = (TOT_CHUNKS + NCH_MAX) * CHUNK
RPS = N_PAD // NS


def _mesh():
    return plsc.VectorSubcoreMesh(core_axis_name="c", subcore_axis_name="s",
                                  num_cores=NC, num_subcores=NS)


def _chunk_range(c, s):
    base = jnp.where(c == 0, s * NCH0, NS * NCH0 + s * NCH1)
    ng = jnp.where(c == 0, NCH0 // NBUF, NCH1 // NBUF)
    return pl.multiple_of(base, 8), ng


def _make_deg_kernel():
    F = 16

    @functools.partial(
        pl.kernel,
        mesh=_mesh(),
        out_type=jax.ShapeDtypeStruct((NC, N_PAD, F), jnp.float32),
        scratch_types=[
            pltpu.VMEM((NCH_MAX, CHUNK), jnp.int32),
            pltpu.VMEM((CHUNK, F), jnp.float32),
            pltpu.VMEM_SHARED((N_PAD, F), jnp.float32),
            pltpu.SemaphoreType.DMA,
        ],
        compiler_params=pltpu.CompilerParams(use_tc_tiling_on_sc=False),
    )
    def deg_kernel(dst_hbm, zeros_hbm, ones_hbm, out_hbm, dst_v, ones_v,
                   acc_sh, sem):
        c = lax.axis_index("c")
        s = lax.axis_index("s")
        base, ng = _chunk_range(c, s)
        off = pl.multiple_of(s * RPS, 8)
        pltpu.sync_copy(zeros_hbm, acc_sh.at[pl.ds(off, RPS)])
        pltpu.sync_copy(ones_hbm, ones_v)
        pltpu.sync_copy(dst_hbm.at[pl.ds(base, NCH_MAX)], dst_v)
        plsc.subcore_barrier()

        def fire(g, _):
            for b in range(NBUF):
                j = g * NBUF + b
                pltpu.async_copy(ones_v, acc_sh.at[dst_v.at[j]], sem,
                                 add=True)
            return ()

        lax.fori_loop(0, ng, fire, (), unroll=False)

        def drain(g, _):
            for b in range(NBUF):
                j = g * NBUF + b
                pltpu.make_async_copy(ones_v, acc_sh.at[dst_v.at[j]],
                                      sem).wait()
            return ()

        lax.fori_loop(0, ng, drain, (), unroll=False)
        plsc.subcore_barrier()
        pltpu.sync_copy(acc_sh.at[pl.ds(off, RPS)], out_hbm.at[c, pl.ds(off, RPS)])

    return deg_kernel


def _make_agg_kernel(F):
    @functools.partial(
        pl.kernel,
        mesh=_mesh(),
        out_type=jax.ShapeDtypeStruct((NC, N_PAD, F), jnp.float32),
        scratch_types=[
            pltpu.VMEM((NCH_MAX, CHUNK), jnp.int32),
            pltpu.VMEM((NCH_MAX, CHUNK), jnp.int32),
            pltpu.VMEM((NBUF, CHUNK, F), jnp.float32),
            pltpu.VMEM_SHARED((N_PAD, F), jnp.float32),
            [pltpu.SemaphoreType.DMA] * NBUF,
            [pltpu.SemaphoreType.DMA] * NBUF,
        ],
        compiler_params=pltpu.CompilerParams(use_tc_tiling_on_sc=False),
    )
    def agg_kernel(g_hbm, src_hbm, dst_hbm, zeros_hbm, out_hbm, src_v, dst_v,
                   rows_v, acc_sh, gsems, ssems):
        c = lax.axis_index("c")
        s = lax.axis_index("s")
        base, ng = _chunk_range(c, s)
        off = pl.multiple_of(s * RPS, 8)
        pltpu.sync_copy(zeros_hbm, acc_sh.at[pl.ds(off, RPS)])
        pltpu.sync_copy(src_hbm.at[pl.ds(base, NCH_MAX)], src_v)
        pltpu.sync_copy(dst_hbm.at[pl.ds(base, NCH_MAX)], dst_v)
        plsc.subcore_barrier()

        def gather(j, b):
            pltpu.async_copy(g_hbm.at[src_v.at[j]], rows_v.at[b], gsems[b])

        def gather_wait(j, b):
            pltpu.make_async_copy(g_hbm.at[src_v.at[j]], rows_v.at[b],
                                  gsems[b]).wait()

        def scat(j, b):
            pltpu.async_copy(rows_v.at[b], acc_sh.at[dst_v.at[j]], ssems[b],
                             add=True)

        def scat_wait(j, b):
            pltpu.make_async_copy(rows_v.at[b], acc_sh.at[dst_v.at[j]],
                                  ssems[b]).wait()

        def slot(j, b, do_sw, do_g):
            gather_wait(j, b)
            scat(j, b)
            bn = (b + 2) % NBUF
            if do_sw:
                scat_wait(j - 2, bn)
            if do_g:
                gather(j + 2, bn)

        gather(0, 0)
        gather(1, 1)
        slot(0, 0, False, True)
        slot(1, 1, False, True)
        slot(2, 2, True, True)
        slot(3, 3, True, True)

        def body(g, _):
            for b in range(NBUF):
                slot(g * NBUF + b, b, True, True)
            return ()

        lax.fori_loop(1, ng - 1, body, (), unroll=False)
        j0 = (ng - 1) * NBUF
        slot(j0 + 0, 0, True, True)
        slot(j0 + 1, 1, True, True)
        slot(j0 + 2, 2, False, False)
        slot(j0 + 3, 3, False, False)
        for b in range(NBUF):
            scat_wait(j0 + b, b)

        plsc.subcore_barrier()
        pltpu.sync_copy(acc_sh.at[pl.ds(off, RPS)], out_hbm.at[c, pl.ds(off, RPS)])

    return agg_kernel


@functools.cache
def _sc_kernels():
    return (_make_deg_kernel(), _make_agg_kernel(64), _make_agg_kernel(32),
            _make_agg_kernel(48))


_BR = 512
_TC_GRID = N_PAD // _BR
_BRF = 400
_TC_GRID_F = N // _BRF


def _dinv_of(degp_ref):
    deg = degp_ref[0, :, 0:1] + degp_ref[1, :, 0:1] + 1.0
    return lax.rsqrt(deg)


def _tc1_body(x_ref, w_ref, degp_ref, h_ref, g_ref):
    h = jnp.dot(x_ref[...], w_ref[...], preferred_element_type=jnp.float32)
    dinv = _dinv_of(degp_ref)
    h_ref[...] = h
    g_ref[...] = h * dinv


def _tc_mid_body(h_ref, degp_ref, accp_ref, b_ref, w_ref, hn_ref, gn_ref):
    dinv = _dinv_of(degp_ref)
    acc = accp_ref[0] + accp_ref[1]
    z = jnp.maximum(acc * dinv + h_ref[...] * (dinv * dinv) + b_ref[...], 0.0)
    hn = jnp.dot(z, w_ref[...], preferred_element_type=jnp.float32)
    hn_ref[...] = hn
    gn_ref[...] = hn * dinv


def _tc_fin_body(h_ref, degp_ref, accp_ref, b_ref, out_ref):
    dinv = _dinv_of(degp_ref)
    acc = accp_ref[0] + accp_ref[1]
    z = acc * dinv + h_ref[...] * (dinv * dinv) + b_ref[...]
    z = z[:, :40]
    m = jnp.max(z, axis=1, keepdims=True)
    e = jnp.exp(z - m)
    out_ref[...] = e / jnp.sum(e, axis=1, keepdims=True)


def _row_spec(r, f):
    return pl.BlockSpec((r, f), lambda i: (i, 0))


def _degp_spec(r):
    return pl.BlockSpec((NC, r, 16), lambda i: (0, i, 0))


def _accp_spec(r, f):
    return pl.BlockSpec((NC, r, f), lambda i: (0, i, 0))


def _full_spec(a, b):
    return pl.BlockSpec((a, b), lambda i: (0, 0))


def _tc1(x, w1, degp):
    return pl.pallas_call(
        _tc1_body,
        grid=(_TC_GRID,),
        in_specs=[_row_spec(_BR, 128), _full_spec(128, 64), _degp_spec(_BR)],
        out_specs=[_row_spec(_BR, 64), _row_spec(_BR, 64)],
        out_shape=[
            jax.ShapeDtypeStruct((N_PAD, 64), jnp.float32),
            jax.ShapeDtypeStruct((N_PAD, 64), jnp.float32),
        ],
    )(x, w1, degp)


def _tc_mid(h, degp, accp, b, w, fin, fout):
    return pl.pallas_call(
        _tc_mid_body,
        grid=(_TC_GRID,),
        in_specs=[_row_spec(_BR, fin), _degp_spec(_BR), _accp_spec(_BR, fin),
                  _full_spec(1, fin), _full_spec(fin, fout)],
        out_specs=[_row_spec(_BR, fout), _row_spec(_BR, fout)],
        out_shape=[
            jax.ShapeDtypeStruct((N_PAD, fout), jnp.float32),
            jax.ShapeDtypeStruct((N_PAD, fout), jnp.float32),
        ],
    )(h, degp, accp, b, w)


def _tc_fin(h, degp, accp, b):
    return pl.pallas_call(
        _tc_fin_body,
        grid=(_TC_GRID_F,),
        in_specs=[_row_spec(_BRF, 48), _degp_spec(_BRF),
                  _accp_spec(_BRF, 48), _full_spec(1, 48)],
        out_specs=_row_spec(_BRF, 40),
        out_shape=jax.ShapeDtypeStruct((N, 40), jnp.float32),
    )(h, degp, accp, b)


def kernel(x, edge_index, W1, b1, W2, b2, W3, b3):
    pad_idx = jnp.full((E_ALLOC - E,), N, jnp.int32)
    src = jnp.concatenate([edge_index[0].astype(jnp.int32), pad_idx])
    dst = jnp.concatenate([edge_index[1].astype(jnp.int32), pad_idx])
    src = src.reshape(TOT_CHUNKS + NCH_MAX, CHUNK)
    dst = dst.reshape(TOT_CHUNKS + NCH_MAX, CHUNK)
    xp = jnp.pad(x, ((0, N_PAD - N), (0, 0)))

    z16 = jnp.zeros((RPS, 16), jnp.float32)
    z64 = jnp.zeros((RPS, 64), jnp.float32)
    z32 = jnp.zeros((RPS, 32), jnp.float32)
    z48 = jnp.zeros((RPS, 48), jnp.float32)
    ones = jnp.ones((CHUNK, 16), jnp.float32)

    W3p = jnp.pad(W3, ((0, 0), (0, 8)))
    b3p = jnp.pad(b3, (0, 8))

    deg_k, agg64, agg32, agg48 = _sc_kernels()
    degp = deg_k(dst, z16, ones)

    h1, g1 = _tc1(xp, W1, degp)
    acc1 = agg64(g1, src, dst, z64)
    h2, g2 = _tc_mid(h1, degp, acc1, b1.reshape(1, 64), W2, 64, 32)
    acc2 = agg32(g2, src, dst, z32)
    h3, g3 = _tc_mid(h2, degp, acc2, b2.reshape(1, 32), W3p, 32, 48)
    acc3 = agg48(g3, src, dst, z48)
    return _tc_fin(h3, degp, acc3, b3p.reshape(1, 48))


# speedup vs baseline: 30.9824x; 1.9213x over previous
import functools

import jax
import jax.numpy as jnp
from jax import lax
from jax.experimental import pallas as pl
from jax.experimental.pallas import tpu as pltpu
from jax.experimental.pallas import tpu_sc as plsc

N = 10000
E = 320000
N_PAD = 10240
NC, NS = 2, 16
CHUNK = 128
NBUF = 4
SPLIT_EVEN = (80, 80)
NCH_MAX = 80
TOT_CHUNKS = NS * 160
E_ALLOC = (TOT_CHUNKS + NCH_MAX) * CHUNK
RPS = N_PAD // NS


def _mesh():
    return plsc.VectorSubcoreMesh(core_axis_name="c", subcore_axis_name="s",
                                  num_cores=NC, num_subcores=NS)


def _chunk_range(c, s, split):
    nch0, nch1 = split
    base = jnp.where(c == 0, s * nch0, NS * nch0 + s * nch1)
    ng = jnp.where(c == 0, nch0 // NBUF, nch1 // NBUF)
    return pl.multiple_of(base, 8), ng


def _make_deg_kernel():
    F = 16

    @functools.partial(
        pl.kernel,
        mesh=_mesh(),
        out_type=jax.ShapeDtypeStruct((NC, N_PAD, F), jnp.float32),
        scratch_types=[
            pltpu.VMEM((NCH_MAX, CHUNK), jnp.int32),
            pltpu.VMEM((CHUNK, F), jnp.float32),
            pltpu.VMEM_SHARED((N_PAD, F), jnp.float32),
            pltpu.SemaphoreType.DMA,
        ],
        compiler_params=pltpu.CompilerParams(use_tc_tiling_on_sc=False),
    )
    def deg_kernel(dst_hbm, zeros_hbm, ones_hbm, out_hbm, dst_v, ones_v,
                   acc_sh, sem):
        c = lax.axis_index("c")
        s = lax.axis_index("s")
        base, ng = _chunk_range(c, s, SPLIT_EVEN)
        off = pl.multiple_of(s * RPS, 8)
        pltpu.sync_copy(zeros_hbm, acc_sh.at[pl.ds(off, RPS)])
        pltpu.sync_copy(ones_hbm, ones_v)
        pltpu.sync_copy(dst_hbm.at[pl.ds(base, NCH_MAX)], dst_v)
        plsc.subcore_barrier()

        def fire(g, _):
            for b in range(NBUF):
                j = g * NBUF + b
                pltpu.async_copy(ones_v, acc_sh.at[dst_v.at[j]], sem,
                                 add=True)
            return ()

        lax.fori_loop(0, ng, fire, (), unroll=False)

        def drain(g, _):
            for b in range(NBUF):
                j = g * NBUF + b
                pltpu.make_async_copy(ones_v, acc_sh.at[dst_v.at[j]],
                                      sem).wait()
            return ()

        lax.fori_loop(0, ng, drain, (), unroll=False)
        plsc.subcore_barrier()
        pltpu.sync_copy(acc_sh.at[pl.ds(off, RPS)], out_hbm.at[c, pl.ds(off, RPS)])

    return deg_kernel


def _make_agg_kernel(F):
    stage = True

    @functools.partial(
        pl.kernel,
        mesh=_mesh(),
        out_type=jax.ShapeDtypeStruct((NC, N_PAD, F), jnp.float32),
        scratch_types=[
            pltpu.VMEM((NCH_MAX, CHUNK), jnp.int32),
            pltpu.VMEM((NCH_MAX, CHUNK), jnp.int32),
            pltpu.VMEM((NBUF, CHUNK, F), jnp.float32),
            pltpu.VMEM_SHARED((N_PAD, F), jnp.float32),
            pltpu.VMEM_SHARED((N_PAD if stage else NBUF, F), jnp.float32),
            [pltpu.SemaphoreType.DMA] * NBUF,
            [pltpu.SemaphoreType.DMA] * NBUF,
        ],
        compiler_params=pltpu.CompilerParams(use_tc_tiling_on_sc=False),
    )
    def agg_kernel(g_hbm, src_hbm, dst_hbm, zeros_hbm, out_hbm, src_v, dst_v,
                   rows_v, acc_sh, g_sh, gsems, ssems):
        c = lax.axis_index("c")
        s = lax.axis_index("s")
        base, ng = _chunk_range(c, s, SPLIT_EVEN)
        off = pl.multiple_of(s * RPS, 8)
        pltpu.sync_copy(zeros_hbm, acc_sh.at[pl.ds(off, RPS)])
        if stage:
            pltpu.sync_copy(g_hbm.at[pl.ds(off, RPS)],
                            g_sh.at[pl.ds(off, RPS)])
        pltpu.sync_copy(src_hbm.at[pl.ds(base, NCH_MAX)], src_v)
        pltpu.sync_copy(dst_hbm.at[pl.ds(base, NCH_MAX)], dst_v)
        plsc.subcore_barrier()
        g_tab = g_sh if stage else g_hbm

        def gather(j, b):
            pltpu.async_copy(g_tab.at[src_v.at[j]], rows_v.at[b], gsems[b])

        def gather_wait(j, b):
            pltpu.make_async_copy(g_tab.at[src_v.at[j]], rows_v.at[b],
                                  gsems[b]).wait()

        def scat(j, b):
            pltpu.async_copy(rows_v.at[b], acc_sh.at[dst_v.at[j]], ssems[b],
                             add=True)

        def scat_wait(j, b):
            pltpu.make_async_copy(rows_v.at[b], acc_sh.at[dst_v.at[j]],
                                  ssems[b]).wait()

        def slot(j, b, do_sw, do_g):
            gather_wait(j, b)
            scat(j, b)
            bn = (b + 2) % NBUF
            if do_sw:
                scat_wait(j - 2, bn)
            if do_g:
                gather(j + 2, bn)

        gather(0, 0)
        gather(1, 1)
        slot(0, 0, False, True)
        slot(1, 1, False, True)
        slot(2, 2, True, True)
        slot(3, 3, True, True)

        def body(g, _):
            for b in range(NBUF):
                slot(g * NBUF + b, b, True, True)
            return ()

        lax.fori_loop(1, ng - 1, body, (), unroll=False)
        j0 = (ng - 1) * NBUF
        slot(j0 + 0, 0, True, True)
        slot(j0 + 1, 1, True, True)
        slot(j0 + 2, 2, False, False)
        slot(j0 + 3, 3, False, False)
        for b in range(NBUF):
            scat_wait(j0 + b, b)

        plsc.subcore_barrier()
        pltpu.sync_copy(acc_sh.at[pl.ds(off, RPS)], out_hbm.at[c, pl.ds(off, RPS)])

    return agg_kernel


@functools.cache
def _sc_kernels():
    return (_make_deg_kernel(), _make_agg_kernel(32), _make_agg_kernel(48))


_BR = 512
_TC_GRID = N_PAD // _BR
_BRF = 400
_TC_GRID_F = N // _BRF


def _dinv_of(degp_ref):
    deg = degp_ref[0, :, 0:1] + degp_ref[1, :, 0:1] + 1.0
    return lax.rsqrt(deg)


def _tc1_body(x_ref, w_ref, degp_ref, h_ref, ga_ref, gb_ref):
    h = jnp.dot(x_ref[...], w_ref[...], preferred_element_type=jnp.float32)
    dinv = _dinv_of(degp_ref)
    g = h * dinv
    h_ref[...] = h
    ga_ref[...] = g[:, :32]
    gb_ref[...] = g[:, 32:]


def _tc_mid2_body(h_ref, degp_ref, acca_ref, accb_ref, b_ref, w_ref, hn_ref,
                  gn_ref):
    dinv = _dinv_of(degp_ref)
    acc = jnp.concatenate([acca_ref[0] + acca_ref[1],
                           accb_ref[0] + accb_ref[1]], axis=1)
    z = jnp.maximum(acc * dinv + h_ref[...] * (dinv * dinv) + b_ref[...], 0.0)
    hn = jnp.dot(z, w_ref[...], preferred_element_type=jnp.float32)
    hn_ref[...] = hn
    gn_ref[...] = hn * dinv


def _tc_mid_body(h_ref, degp_ref, accp_ref, b_ref, w_ref, hn_ref, gn_ref):
    dinv = _dinv_of(degp_ref)
    acc = accp_ref[0] + accp_ref[1]
    z = jnp.maximum(acc * dinv + h_ref[...] * (dinv * dinv) + b_ref[...], 0.0)
    hn = jnp.dot(z, w_ref[...], preferred_element_type=jnp.float32)
    hn_ref[...] = hn
    gn_ref[...] = hn * dinv


def _tc_fin_body(h_ref, degp_ref, accp_ref, b_ref, out_ref):
    dinv = _dinv_of(degp_ref)
    acc = accp_ref[0] + accp_ref[1]
    z = acc * dinv + h_ref[...] * (dinv * dinv) + b_ref[...]
    z = z[:, :40]
    m = jnp.max(z, axis=1, keepdims=True)
    e = jnp.exp(z - m)
    out_ref[...] = e / jnp.sum(e, axis=1, keepdims=True)


def _row_spec(r, f):
    return pl.BlockSpec((r, f), lambda i: (i, 0))


def _degp_spec(r):
    return pl.BlockSpec((NC, r, 16), lambda i: (0, i, 0))


def _accp_spec(r, f):
    return pl.BlockSpec((NC, r, f), lambda i: (0, i, 0))


def _full_spec(a, b):
    return pl.BlockSpec((a, b), lambda i: (0, 0))


def _tc1(x, w1, degp):
    return pl.pallas_call(
        _tc1_body,
        grid=(_TC_GRID,),
        in_specs=[_row_spec(_BR, 128), _full_spec(128, 64), _degp_spec(_BR)],
        out_specs=[_row_spec(_BR, 64), _row_spec(_BR, 32),
                   _row_spec(_BR, 32)],
        out_shape=[
            jax.ShapeDtypeStruct((N_PAD, 64), jnp.float32),
            jax.ShapeDtypeStruct((N_PAD, 32), jnp.float32),
            jax.ShapeDtypeStruct((N_PAD, 32), jnp.float32),
        ],
    )(x, w1, degp)


def _tc_mid2(h, degp, acca, accb, b, w):
    return pl.pallas_call(
        _tc_mid2_body,
        grid=(_TC_GRID,),
        in_specs=[_row_spec(_BR, 64), _degp_spec(_BR), _accp_spec(_BR, 32),
                  _accp_spec(_BR, 32), _full_spec(1, 64), _full_spec(64, 32)],
        out_specs=[_row_spec(_BR, 32), _row_spec(_BR, 32)],
        out_shape=[
            jax.ShapeDtypeStruct((N_PAD, 32), jnp.float32),
            jax.ShapeDtypeStruct((N_PAD, 32), jnp.float32),
        ],
    )(h, degp, acca, accb, b, w)


def _tc_mid(h, degp, accp, b, w, fin, fout):
    return pl.pallas_call(
        _tc_mid_body,
        grid=(_TC_GRID,),
        in_specs=[_row_spec(_BR, fin), _degp_spec(_BR), _accp_spec(_BR, fin),
                  _full_spec(1, fin), _full_spec(fin, fout)],
        out_specs=[_row_spec(_BR, fout), _row_spec(_BR, fout)],
        out_shape=[
            jax.ShapeDtypeStruct((N_PAD, fout), jnp.float32),
            jax.ShapeDtypeStruct((N_PAD, fout), jnp.float32),
        ],
    )(h, degp, accp, b, w)


def _tc_fin(h, degp, accp, b):
    return pl.pallas_call(
        _tc_fin_body,
        grid=(_TC_GRID_F,),
        in_specs=[_row_spec(_BRF, 48), _degp_spec(_BRF),
                  _accp_spec(_BRF, 48), _full_spec(1, 48)],
        out_specs=_row_spec(_BRF, 40),
        out_shape=jax.ShapeDtypeStruct((N, 40), jnp.float32),
    )(h, degp, accp, b)


def kernel(x, edge_index, W1, b1, W2, b2, W3, b3):
    pad_idx = jnp.full((E_ALLOC - E,), N, jnp.int32)
    src = jnp.concatenate([edge_index[0].astype(jnp.int32), pad_idx])
    dst = jnp.concatenate([edge_index[1].astype(jnp.int32), pad_idx])
    src = src.reshape(TOT_CHUNKS + NCH_MAX, CHUNK)
    dst = dst.reshape(TOT_CHUNKS + NCH_MAX, CHUNK)
    xp = jnp.pad(x, ((0, N_PAD - N), (0, 0)))

    z16 = jnp.zeros((RPS, 16), jnp.float32)
    z32 = jnp.zeros((RPS, 32), jnp.float32)
    z48 = jnp.zeros((RPS, 48), jnp.float32)
    ones = jnp.ones((CHUNK, 16), jnp.float32)

    W3p = jnp.pad(W3, ((0, 0), (0, 8)))
    b3p = jnp.pad(b3, (0, 8))

    deg_k, agg32, agg48 = _sc_kernels()
    degp = deg_k(dst, z16, ones)

    h1, g1a, g1b = _tc1(xp, W1, degp)
    acc1a = agg32(g1a, src, dst, z32)
    acc1b = agg32(g1b, src, dst, z32)
    h2, g2 = _tc_mid2(h1, degp, acc1a, acc1b, b1.reshape(1, 64), W2)
    acc2 = agg32(g2, src, dst, z32)
    h3, g3 = _tc_mid(h2, degp, acc2, b2.reshape(1, 32), W3p, 32, 48)
    acc3 = agg48(g3, src, dst, z48)
    return _tc_fin(h3, degp, acc3, b3p.reshape(1, 48))


# speedup vs baseline: 32.8462x; 1.0602x over previous
import functools

import jax
import jax.numpy as jnp
from jax import lax
from jax.experimental import pallas as pl
from jax.experimental.pallas import tpu as pltpu
from jax.experimental.pallas import tpu_sc as plsc

N = 10000
E = 320000
N_PAD = 10240
NC, NS = 2, 16
CHUNK = 128
NBUF = 4
SPLIT_EVEN = (80, 80)
NCH_ALL = 160
NCH_MAX = NCH_ALL
TOT_CHUNKS = NS * 160
E_ALLOC = (TOT_CHUNKS + NCH_MAX) * CHUNK
RPS = N_PAD // NS


def _mesh():
    return plsc.VectorSubcoreMesh(core_axis_name="c", subcore_axis_name="s",
                                  num_cores=NC, num_subcores=NS)


def _chunk_range(c, s, split):
    nch0, nch1 = split
    base = jnp.where(c == 0, s * nch0, NS * nch0 + s * nch1)
    ng = jnp.where(c == 0, nch0 // NBUF, nch1 // NBUF)
    return pl.multiple_of(base, 8), ng


def _make_deg_kernel():
    F = 16

    @functools.partial(
        pl.kernel,
        mesh=_mesh(),
        out_type=jax.ShapeDtypeStruct((NC, N_PAD, F), jnp.float32),
        scratch_types=[
            pltpu.VMEM((NCH_MAX, CHUNK), jnp.int32),
            pltpu.VMEM((CHUNK, F), jnp.float32),
            pltpu.VMEM_SHARED((N_PAD, F), jnp.float32),
            pltpu.SemaphoreType.DMA,
        ],
        compiler_params=pltpu.CompilerParams(use_tc_tiling_on_sc=False),
    )
    def deg_kernel(dst_hbm, zeros_hbm, ones_hbm, out_hbm, dst_v, ones_v,
                   acc_sh, sem):
        c = lax.axis_index("c")
        s = lax.axis_index("s")
        base, ng = _chunk_range(c, s, SPLIT_EVEN)
        off = pl.multiple_of(s * RPS, 8)
        pltpu.sync_copy(zeros_hbm, acc_sh.at[pl.ds(off, RPS)])
        pltpu.sync_copy(ones_hbm, ones_v)
        pltpu.sync_copy(dst_hbm.at[pl.ds(base, NCH_MAX)], dst_v)
        plsc.subcore_barrier()

        def fire(g, _):
            for b in range(NBUF):
                j = g * NBUF + b
                pltpu.async_copy(ones_v, acc_sh.at[dst_v.at[j]], sem,
                                 add=True)
            return ()

        lax.fori_loop(0, ng, fire, (), unroll=False)

        def drain(g, _):
            for b in range(NBUF):
                j = g * NBUF + b
                pltpu.make_async_copy(ones_v, acc_sh.at[dst_v.at[j]],
                                      sem).wait()
            return ()

        lax.fori_loop(0, ng, drain, (), unroll=False)
        plsc.subcore_barrier()
        pltpu.sync_copy(acc_sh.at[pl.ds(off, RPS)], out_hbm.at[c, pl.ds(off, RPS)])

    return deg_kernel


def _make_agg_kernel(F):
    stage = True

    @functools.partial(
        pl.kernel,
        mesh=_mesh(),
        out_type=jax.ShapeDtypeStruct((NC, N_PAD, F), jnp.float32),
        scratch_types=[
            pltpu.VMEM((NCH_MAX, CHUNK), jnp.int32),
            pltpu.VMEM((NCH_MAX, CHUNK), jnp.int32),
            pltpu.VMEM((NBUF, CHUNK, F), jnp.float32),
            pltpu.VMEM_SHARED((N_PAD, F), jnp.float32),
            pltpu.VMEM_SHARED((N_PAD if stage else NBUF, F), jnp.float32),
            [pltpu.SemaphoreType.DMA] * NBUF,
            [pltpu.SemaphoreType.DMA] * NBUF,
        ],
        compiler_params=pltpu.CompilerParams(use_tc_tiling_on_sc=False),
    )
    def agg_kernel(g_hbm, src_hbm, dst_hbm, zeros_hbm, out_hbm, src_v, dst_v,
                   rows_v, acc_sh, g_sh, gsems, ssems):
        c = lax.axis_index("c")
        s = lax.axis_index("s")
        base, ng = _chunk_range(c, s, SPLIT_EVEN)
        off = pl.multiple_of(s * RPS, 8)
        pltpu.sync_copy(zeros_hbm, acc_sh.at[pl.ds(off, RPS)])
        if stage:
            pltpu.sync_copy(g_hbm.at[pl.ds(off, RPS)],
                            g_sh.at[pl.ds(off, RPS)])
        pltpu.sync_copy(src_hbm.at[pl.ds(base, NCH_MAX)], src_v)
        pltpu.sync_copy(dst_hbm.at[pl.ds(base, NCH_MAX)], dst_v)
        plsc.subcore_barrier()
        _edge_pipeline(g_sh, acc_sh, src_v, dst_v, rows_v, gsems, ssems, ng)
        plsc.subcore_barrier()
        pltpu.sync_copy(acc_sh.at[pl.ds(off, RPS)], out_hbm.at[c, pl.ds(off, RPS)])

    return agg_kernel


def _edge_pipeline(g_tab, acc_sh, src_v, dst_v, rows_v, gsems, ssems, ng):
    def gather(j, b):
        pltpu.async_copy(g_tab.at[src_v.at[j]], rows_v.at[b], gsems[b])

    def gather_wait(j, b):
        pltpu.make_async_copy(g_tab.at[src_v.at[j]], rows_v.at[b],
                              gsems[b]).wait()

    def scat(j, b):
        pltpu.async_copy(rows_v.at[b], acc_sh.at[dst_v.at[j]], ssems[b],
                         add=True)

    def scat_wait(j, b):
        pltpu.make_async_copy(rows_v.at[b], acc_sh.at[dst_v.at[j]],
                              ssems[b]).wait()

    def slot(j, b, do_sw, do_g):
        gather_wait(j, b)
        scat(j, b)
        bn = (b + 2) % NBUF
        if do_sw:
            scat_wait(j - 2, bn)
        if do_g:
            gather(j + 2, bn)

    gather(0, 0)
    gather(1, 1)
    slot(0, 0, False, True)
    slot(1, 1, False, True)
    slot(2, 2, True, True)
    slot(3, 3, True, True)

    def body(g, _):
        for b in range(NBUF):
            slot(g * NBUF + b, b, True, True)
        return ()

    lax.fori_loop(1, ng - 1, body, (), unroll=False)
    j0 = (ng - 1) * NBUF
    slot(j0 + 0, 0, True, True)
    slot(j0 + 1, 1, True, True)
    slot(j0 + 2, 2, False, False)
    slot(j0 + 3, 3, False, False)
    for b in range(NBUF):
        scat_wait(j0 + b, b)


def _make_dual_agg_kernel(F):
    @functools.partial(
        pl.kernel,
        mesh=_mesh(),
        out_type=[
            jax.ShapeDtypeStruct((N_PAD, F), jnp.float32),
            jax.ShapeDtypeStruct((N_PAD, F), jnp.float32),
        ],
        scratch_types=[
            pltpu.VMEM((NCH_ALL, CHUNK), jnp.int32),
            pltpu.VMEM((NCH_ALL, CHUNK), jnp.int32),
            pltpu.VMEM((NBUF, CHUNK, F), jnp.float32),
            pltpu.VMEM_SHARED((N_PAD, F), jnp.float32),
            pltpu.VMEM_SHARED((N_PAD, F), jnp.float32),
            [pltpu.SemaphoreType.DMA] * NBUF,
            [pltpu.SemaphoreType.DMA] * NBUF,
        ],
        compiler_params=pltpu.CompilerParams(use_tc_tiling_on_sc=False),
    )
    def dual_kernel(ga_hbm, gb_hbm, src_hbm, dst_hbm, zeros_hbm, outa_hbm,
                    outb_hbm, src_v, dst_v, rows_v, acc_sh, g_sh, gsems,
                    ssems):
        c = lax.axis_index("c")
        s = lax.axis_index("s")
        base = pl.multiple_of(s * NCH_ALL, 8)
        ng = NCH_ALL // NBUF
        off = pl.multiple_of(s * RPS, 8)
        pltpu.sync_copy(zeros_hbm, acc_sh.at[pl.ds(off, RPS)])

        @pl.when(c == 0)
        def _():
            pltpu.sync_copy(ga_hbm.at[pl.ds(off, RPS)],
                            g_sh.at[pl.ds(off, RPS)])

        @pl.when(c == 1)
        def _():
            pltpu.sync_copy(gb_hbm.at[pl.ds(off, RPS)],
                            g_sh.at[pl.ds(off, RPS)])

        pltpu.sync_copy(src_hbm.at[pl.ds(base, NCH_ALL)], src_v)
        pltpu.sync_copy(dst_hbm.at[pl.ds(base, NCH_ALL)], dst_v)
        plsc.subcore_barrier()
        _edge_pipeline(g_sh, acc_sh, src_v, dst_v, rows_v, gsems, ssems, ng)
        plsc.subcore_barrier()

        @pl.when(c == 0)
        def _():
            pltpu.sync_copy(acc_sh.at[pl.ds(off, RPS)],
                            outa_hbm.at[pl.ds(off, RPS)])

        @pl.when(c == 1)
        def _():
            pltpu.sync_copy(acc_sh.at[pl.ds(off, RPS)],
                            outb_hbm.at[pl.ds(off, RPS)])

    return dual_kernel


@functools.cache
def _sc_kernels():
    return (_make_deg_kernel(), _make_agg_kernel(32), _make_agg_kernel(48),
            _make_dual_agg_kernel(32))


_BR = 512
_TC_GRID = N_PAD // _BR
_BRF = 400
_TC_GRID_F = N // _BRF


def _dinv_of(degp_ref):
    deg = degp_ref[0, :, 0:1] + degp_ref[1, :, 0:1] + 1.0
    return lax.rsqrt(deg)


def _tc_mm_body(x_ref, w_ref, h_ref):
    h_ref[...] = jnp.dot(x_ref[...], w_ref[...],
                         preferred_element_type=jnp.float32)


def _tc_scale_body(h_ref, degp_ref, ga_ref, gb_ref):
    dinv = _dinv_of(degp_ref)
    g = h_ref[...] * dinv
    ga_ref[...] = g[:, :32]
    gb_ref[...] = g[:, 32:]


def _tc_mid2_body(h_ref, degp_ref, acca_ref, accb_ref, b_ref, w_ref, hn_ref,
                  gn_ref):
    dinv = _dinv_of(degp_ref)
    acc = jnp.concatenate([acca_ref[...], accb_ref[...]], axis=1)
    z = jnp.maximum(acc * dinv + h_ref[...] * (dinv * dinv) + b_ref[...], 0.0)
    hn = jnp.dot(z, w_ref[...], preferred_element_type=jnp.float32)
    hn_ref[...] = hn
    gn_ref[...] = hn * dinv


def _tc_mid_body(h_ref, degp_ref, accp_ref, b_ref, w_ref, hn_ref, gn_ref):
    dinv = _dinv_of(degp_ref)
    acc = accp_ref[0] + accp_ref[1]
    z = jnp.maximum(acc * dinv + h_ref[...] * (dinv * dinv) + b_ref[...], 0.0)
    hn = jnp.dot(z, w_ref[...], preferred_element_type=jnp.float32)
    hn_ref[...] = hn
    gn_ref[...] = hn * dinv


def _tc_fin_body(h_ref, degp_ref, accp_ref, b_ref, out_ref):
    dinv = _dinv_of(degp_ref)
    acc = accp_ref[0] + accp_ref[1]
    z = acc * dinv + h_ref[...] * (dinv * dinv) + b_ref[...]
    z = z[:, :40]
    m = jnp.max(z, axis=1, keepdims=True)
    e = jnp.exp(z - m)
    out_ref[...] = e / jnp.sum(e, axis=1, keepdims=True)


def _row_spec(r, f):
    return pl.BlockSpec((r, f), lambda i: (i, 0))


def _degp_spec(r):
    return pl.BlockSpec((NC, r, 16), lambda i: (0, i, 0))


def _accp_spec(r, f):
    return pl.BlockSpec((NC, r, f), lambda i: (0, i, 0))


def _full_spec(a, b):
    return pl.BlockSpec((a, b), lambda i: (0, 0))


def _tc_mm(x, w1):
    return pl.pallas_call(
        _tc_mm_body,
        grid=(_TC_GRID,),
        in_specs=[_row_spec(_BR, 128), _full_spec(128, 64)],
        out_specs=_row_spec(_BR, 64),
        out_shape=jax.ShapeDtypeStruct((N_PAD, 64), jnp.float32),
    )(x, w1)


def _tc_scale(h, degp):
    return pl.pallas_call(
        _tc_scale_body,
        grid=(_TC_GRID,),
        in_specs=[_row_spec(_BR, 64), _degp_spec(_BR)],
        out_specs=[_row_spec(_BR, 32), _row_spec(_BR, 32)],
        out_shape=[
            jax.ShapeDtypeStruct((N_PAD, 32), jnp.float32),
            jax.ShapeDtypeStruct((N_PAD, 32), jnp.float32),
        ],
    )(h, degp)


def _tc_mid2(h, degp, acca, accb, b, w):
    return pl.pallas_call(
        _tc_mid2_body,
        grid=(_TC_GRID,),
        in_specs=[_row_spec(_BR, 64), _degp_spec(_BR), _row_spec(_BR, 32),
                  _row_spec(_BR, 32), _full_spec(1, 64), _full_spec(64, 32)],
        out_specs=[_row_spec(_BR, 32), _row_spec(_BR, 32)],
        out_shape=[
            jax.ShapeDtypeStruct((N_PAD, 32), jnp.float32),
            jax.ShapeDtypeStruct((N_PAD, 32), jnp.float32),
        ],
    )(h, degp, acca, accb, b, w)


def _tc_mid(h, degp, accp, b, w, fin, fout):
    return pl.pallas_call(
        _tc_mid_body,
        grid=(_TC_GRID,),
        in_specs=[_row_spec(_BR, fin), _degp_spec(_BR), _accp_spec(_BR, fin),
                  _full_spec(1, fin), _full_spec(fin, fout)],
        out_specs=[_row_spec(_BR, fout), _row_spec(_BR, fout)],
        out_shape=[
            jax.ShapeDtypeStruct((N_PAD, fout), jnp.float32),
            jax.ShapeDtypeStruct((N_PAD, fout), jnp.float32),
        ],
    )(h, degp, accp, b, w)


def _tc_fin(h, degp, accp, b):
    return pl.pallas_call(
        _tc_fin_body,
        grid=(_TC_GRID_F,),
        in_specs=[_row_spec(_BRF, 48), _degp_spec(_BRF),
                  _accp_spec(_BRF, 48), _full_spec(1, 48)],
        out_specs=_row_spec(_BRF, 40),
        out_shape=jax.ShapeDtypeStruct((N, 40), jnp.float32),
    )(h, degp, accp, b)


def kernel(x, edge_index, W1, b1, W2, b2, W3, b3):
    pad_idx = jnp.full((E_ALLOC - E,), N, jnp.int32)
    src = jnp.concatenate([edge_index[0].astype(jnp.int32), pad_idx])
    dst = jnp.concatenate([edge_index[1].astype(jnp.int32), pad_idx])
    src = src.reshape(TOT_CHUNKS + NCH_MAX, CHUNK)
    dst = dst.reshape(TOT_CHUNKS + NCH_MAX, CHUNK)
    xp = jnp.pad(x, ((0, N_PAD - N), (0, 0)))

    z16 = jnp.zeros((RPS, 16), jnp.float32)
    z32 = jnp.zeros((RPS, 32), jnp.float32)
    z48 = jnp.zeros((RPS, 48), jnp.float32)
    ones = jnp.ones((CHUNK, 16), jnp.float32)

    W3p = jnp.pad(W3, ((0, 0), (0, 8)))
    b3p = jnp.pad(b3, (0, 8))

    deg_k, agg32, agg48, dual32 = _sc_kernels()
    degp = deg_k(dst, z16, ones)

    h1 = _tc_mm(xp, W1)
    g1a, g1b = _tc_scale(h1, degp)
    acc1a, acc1b = dual32(g1a, g1b, src, dst, z32)
    h2, g2 = _tc_mid2(h1, degp, acc1a, acc1b, b1.reshape(1, 64), W2)
    acc2 = agg32(g2, src, dst, z32)
    h3, g3 = _tc_mid(h2, degp, acc2, b2.reshape(1, 32), W3p, 32, 48)
    acc3 = agg48(g3, src, dst, z48)
    return _tc_fin(h3, degp, acc3, b3p.reshape(1, 48))


# speedup vs baseline: 35.1559x; 1.0703x over previous
import functools

import jax
import jax.numpy as jnp
from jax import lax
from jax.experimental import pallas as pl
from jax.experimental.pallas import tpu as pltpu
from jax.experimental.pallas import tpu_sc as plsc

N = 10000
E = 320000
N_PAD = 10240
NC, NS = 2, 16
CHUNK = 128
NBUF = 4
SPLIT_EVEN = (80, 80)
NCH_ALL = 160
NCH_MAX = NCH_ALL
TOT_CHUNKS = NS * 160
E_ALLOC = (TOT_CHUNKS + NCH_MAX) * CHUNK
RPS = N_PAD // NS


def _mesh():
    return plsc.VectorSubcoreMesh(core_axis_name="c", subcore_axis_name="s",
                                  num_cores=NC, num_subcores=NS)


def _chunk_range(c, s, split):
    nch0, nch1 = split
    base = jnp.where(c == 0, s * nch0, NS * nch0 + s * nch1)
    ng = jnp.where(c == 0, nch0 // NBUF, nch1 // NBUF)
    return pl.multiple_of(base, 8), ng


def _make_deg_kernel():
    F = 16

    @functools.partial(
        pl.kernel,
        mesh=_mesh(),
        out_type=jax.ShapeDtypeStruct((NC, N_PAD, F), jnp.float32),
        scratch_types=[
            pltpu.VMEM((NCH_MAX, CHUNK), jnp.int32),
            pltpu.VMEM((CHUNK, F), jnp.float32),
            pltpu.VMEM_SHARED((N_PAD, F), jnp.float32),
            pltpu.SemaphoreType.DMA,
        ],
        compiler_params=pltpu.CompilerParams(use_tc_tiling_on_sc=False),
    )
    def deg_kernel(dst_hbm, zeros_hbm, ones_hbm, out_hbm, dst_v, ones_v,
                   acc_sh, sem):
        c = lax.axis_index("c")
        s = lax.axis_index("s")
        base, ng = _chunk_range(c, s, SPLIT_EVEN)
        off = pl.multiple_of(s * RPS, 8)
        pltpu.sync_copy(zeros_hbm, acc_sh.at[pl.ds(off, RPS)])
        pltpu.sync_copy(ones_hbm, ones_v)
        pltpu.sync_copy(dst_hbm.at[pl.ds(base, NCH_MAX)], dst_v)
        plsc.subcore_barrier()

        def fire(g, _):
            for b in range(NBUF):
                j = g * NBUF + b
                pltpu.async_copy(ones_v, acc_sh.at[dst_v.at[j]], sem,
                                 add=True)
            return ()

        lax.fori_loop(0, ng, fire, (), unroll=False)

        def drain(g, _):
            for b in range(NBUF):
                j = g * NBUF + b
                pltpu.make_async_copy(ones_v, acc_sh.at[dst_v.at[j]],
                                      sem).wait()
            return ()

        lax.fori_loop(0, ng, drain, (), unroll=False)
        plsc.subcore_barrier()
        pltpu.sync_copy(acc_sh.at[pl.ds(off, RPS)], out_hbm.at[c, pl.ds(off, RPS)])

    return deg_kernel


def _make_agg_kernel(F):
    stage = True

    @functools.partial(
        pl.kernel,
        mesh=_mesh(),
        out_type=jax.ShapeDtypeStruct((NC, N_PAD, F), jnp.float32),
        scratch_types=[
            pltpu.VMEM((NCH_MAX, CHUNK), jnp.int32),
            pltpu.VMEM((NCH_MAX, CHUNK), jnp.int32),
            pltpu.VMEM((NBUF, CHUNK, F), jnp.float32),
            pltpu.VMEM_SHARED((N_PAD, F), jnp.float32),
            pltpu.VMEM_SHARED((N_PAD if stage else NBUF, F), jnp.float32),
            [pltpu.SemaphoreType.DMA] * NBUF,
            [pltpu.SemaphoreType.DMA] * NBUF,
        ],
        compiler_params=pltpu.CompilerParams(use_tc_tiling_on_sc=False),
    )
    def agg_kernel(g_hbm, src_hbm, dst_hbm, zeros_hbm, out_hbm, src_v, dst_v,
                   rows_v, acc_sh, g_sh, gsems, ssems):
        c = lax.axis_index("c")
        s = lax.axis_index("s")
        base, ng = _chunk_range(c, s, SPLIT_EVEN)
        off = pl.multiple_of(s * RPS, 8)
        pltpu.sync_copy(zeros_hbm, acc_sh.at[pl.ds(off, RPS)])
        if stage:
            pltpu.sync_copy(g_hbm.at[pl.ds(off, RPS)],
                            g_sh.at[pl.ds(off, RPS)])
        pltpu.sync_copy(src_hbm.at[pl.ds(base, NCH_MAX)], src_v)
        pltpu.sync_copy(dst_hbm.at[pl.ds(base, NCH_MAX)], dst_v)
        plsc.subcore_barrier()
        _edge_pipeline(g_sh, acc_sh, src_v, dst_v, rows_v, gsems, ssems, ng)
        plsc.subcore_barrier()
        pltpu.sync_copy(acc_sh.at[pl.ds(off, RPS)], out_hbm.at[c, pl.ds(off, RPS)])

    return agg_kernel


def _edge_pipeline(g_tab, acc_sh, src_v, dst_v, rows_v, gsems, ssems, ng):
    def gather(j, b):
        pltpu.async_copy(g_tab.at[src_v.at[j]], rows_v.at[b], gsems[b])

    def gather_wait(j, b):
        pltpu.make_async_copy(g_tab.at[src_v.at[j]], rows_v.at[b],
                              gsems[b]).wait()

    def scat(j, b):
        pltpu.async_copy(rows_v.at[b], acc_sh.at[dst_v.at[j]], ssems[b],
                         add=True)

    def scat_wait(j, b):
        pltpu.make_async_copy(rows_v.at[b], acc_sh.at[dst_v.at[j]],
                              ssems[b]).wait()

    def slot(j, b, do_sw, do_g):
        gather_wait(j, b)
        scat(j, b)
        bn = (b + 2) % NBUF
        if do_sw:
            scat_wait(j - 2, bn)
        if do_g:
            gather(j + 2, bn)

    gather(0, 0)
    gather(1, 1)
    slot(0, 0, False, True)
    slot(1, 1, False, True)
    slot(2, 2, True, True)
    slot(3, 3, True, True)

    def body(g, _):
        for b in range(NBUF):
            slot(g * NBUF + b, b, True, True)
        return ()

    lax.fori_loop(1, ng - 1, body, (), unroll=False)
    j0 = (ng - 1) * NBUF
    slot(j0 + 0, 0, True, True)
    slot(j0 + 1, 1, True, True)
    slot(j0 + 2, 2, False, False)
    slot(j0 + 3, 3, False, False)
    for b in range(NBUF):
        scat_wait(j0 + b, b)


def _make_dual_agg_kernel(F):
    @functools.partial(
        pl.kernel,
        mesh=_mesh(),
        out_type=[
            jax.ShapeDtypeStruct((N_PAD, F), jnp.float32),
            jax.ShapeDtypeStruct((N_PAD, F), jnp.float32),
        ],
        scratch_types=[
            pltpu.VMEM((NCH_ALL, CHUNK), jnp.int32),
            pltpu.VMEM((NCH_ALL, CHUNK), jnp.int32),
            pltpu.VMEM((NBUF, CHUNK, F), jnp.float32),
            pltpu.VMEM_SHARED((N_PAD, F), jnp.float32),
            pltpu.VMEM_SHARED((N_PAD, F), jnp.float32),
            [pltpu.SemaphoreType.DMA] * NBUF,
            [pltpu.SemaphoreType.DMA] * NBUF,
        ],
        compiler_params=pltpu.CompilerParams(use_tc_tiling_on_sc=False),
    )
    def dual_kernel(ga_hbm, gb_hbm, src_hbm, dst_hbm, zeros_hbm, outa_hbm,
                    outb_hbm, src_v, dst_v, rows_v, acc_sh, g_sh, gsems,
                    ssems):
        c = lax.axis_index("c")
        s = lax.axis_index("s")
        base = pl.multiple_of(s * NCH_ALL, 8)
        ng = NCH_ALL // NBUF
        off = pl.multiple_of(s * RPS, 8)
        pltpu.sync_copy(zeros_hbm, acc_sh.at[pl.ds(off, RPS)])

        @pl.when(c == 0)
        def _():
            pltpu.sync_copy(ga_hbm.at[pl.ds(off, RPS)],
                            g_sh.at[pl.ds(off, RPS)])

        @pl.when(c == 1)
        def _():
            pltpu.sync_copy(gb_hbm.at[pl.ds(off, RPS)],
                            g_sh.at[pl.ds(off, RPS)])

        pltpu.sync_copy(src_hbm.at[pl.ds(base, NCH_ALL)], src_v)
        pltpu.sync_copy(dst_hbm.at[pl.ds(base, NCH_ALL)], dst_v)
        plsc.subcore_barrier()
        _edge_pipeline(g_sh, acc_sh, src_v, dst_v, rows_v, gsems, ssems, ng)
        plsc.subcore_barrier()

        @pl.when(c == 0)
        def _():
            pltpu.sync_copy(acc_sh.at[pl.ds(off, RPS)],
                            outa_hbm.at[pl.ds(off, RPS)])

        @pl.when(c == 1)
        def _():
            pltpu.sync_copy(acc_sh.at[pl.ds(off, RPS)],
                            outb_hbm.at[pl.ds(off, RPS)])

    return dual_kernel


@functools.cache
def _sc_kernels():
    return (_make_deg_kernel(), _make_agg_kernel(32), _make_agg_kernel(48),
            _make_dual_agg_kernel(32))


_BR = 1024
_TC_GRID = N_PAD // _BR
_BRF = 2000
_TC_GRID_F = N // _BRF


def _dinv_of(degp_ref):
    deg = degp_ref[0, :, 0:1] + degp_ref[1, :, 0:1] + 1.0
    return lax.rsqrt(deg)


def _tc1_body(x_ref, w_ref, degp_ref, h_ref, ga_ref, gb_ref):
    h = jnp.dot(x_ref[...], w_ref[...], preferred_element_type=jnp.float32)
    dinv = _dinv_of(degp_ref)
    g = h * dinv
    h_ref[...] = h
    ga_ref[...] = g[:, :32]
    gb_ref[...] = g[:, 32:]


def _tc_mid2_body(h_ref, degp_ref, acca_ref, accb_ref, b_ref, w_ref, hn_ref,
                  gn_ref):
    dinv = _dinv_of(degp_ref)
    acc = jnp.concatenate([acca_ref[...], accb_ref[...]], axis=1)
    z = jnp.maximum(acc * dinv + h_ref[...] * (dinv * dinv) + b_ref[...], 0.0)
    hn = jnp.dot(z, w_ref[...], preferred_element_type=jnp.float32)
    hn_ref[...] = hn
    gn_ref[...] = hn * dinv


def _tc_mid_body(h_ref, degp_ref, accp_ref, b_ref, w_ref, hn_ref, gn_ref):
    dinv = _dinv_of(degp_ref)
    acc = accp_ref[0] + accp_ref[1]
    z = jnp.maximum(acc * dinv + h_ref[...] * (dinv * dinv) + b_ref[...], 0.0)
    hn = jnp.dot(z, w_ref[...], preferred_element_type=jnp.float32)
    hn_ref[...] = hn
    gn_ref[...] = hn * dinv


def _tc_fin_body(h_ref, degp_ref, accp_ref, b_ref, out_ref):
    dinv = _dinv_of(degp_ref)
    acc = accp_ref[0] + accp_ref[1]
    z = acc * dinv + h_ref[...] * (dinv * dinv) + b_ref[...]
    z = z[:, :40]
    m = jnp.max(z, axis=1, keepdims=True)
    e = jnp.exp(z - m)
    out_ref[...] = e / jnp.sum(e, axis=1, keepdims=True)


def _row_spec(r, f):
    return pl.BlockSpec((r, f), lambda i: (i, 0))


def _degp_spec(r):
    return pl.BlockSpec((NC, r, 16), lambda i: (0, i, 0))


def _accp_spec(r, f):
    return pl.BlockSpec((NC, r, f), lambda i: (0, i, 0))


def _full_spec(a, b):
    return pl.BlockSpec((a, b), lambda i: (0, 0))


def _tc1(x, w1, degp):
    return pl.pallas_call(
        _tc1_body,
        grid=(_TC_GRID,),
        in_specs=[_row_spec(_BR, 128), _full_spec(128, 64), _degp_spec(_BR)],
        out_specs=[_row_spec(_BR, 64), _row_spec(_BR, 32),
                   _row_spec(_BR, 32)],
        out_shape=[
            jax.ShapeDtypeStruct((N_PAD, 64), jnp.float32),
            jax.ShapeDtypeStruct((N_PAD, 32), jnp.float32),
            jax.ShapeDtypeStruct((N_PAD, 32), jnp.float32),
        ],
    )(x, w1, degp)


def _tc_mid2(h, degp, acca, accb, b, w):
    return pl.pallas_call(
        _tc_mid2_body,
        grid=(_TC_GRID,),
        in_specs=[_row_spec(_BR, 64), _degp_spec(_BR), _row_spec(_BR, 32),
                  _row_spec(_BR, 32), _full_spec(1, 64), _full_spec(64, 32)],
        out_specs=[_row_spec(_BR, 32), _row_spec(_BR, 32)],
        out_shape=[
            jax.ShapeDtypeStruct((N_PAD, 32), jnp.float32),
            jax.ShapeDtypeStruct((N_PAD, 32), jnp.float32),
        ],
    )(h, degp, acca, accb, b, w)


def _tc_mid(h, degp, accp, b, w, fin, fout):
    return pl.pallas_call(
        _tc_mid_body,
        grid=(_TC_GRID,),
        in_specs=[_row_spec(_BR, fin), _degp_spec(_BR), _accp_spec(_BR, fin),
                  _full_spec(1, fin), _full_spec(fin, fout)],
        out_specs=[_row_spec(_BR, fout), _row_spec(_BR, fout)],
        out_shape=[
            jax.ShapeDtypeStruct((N_PAD, fout), jnp.float32),
            jax.ShapeDtypeStruct((N_PAD, fout), jnp.float32),
        ],
    )(h, degp, accp, b, w)


def _tc_fin(h, degp, accp, b):
    return pl.pallas_call(
        _tc_fin_body,
        grid=(_TC_GRID_F,),
        in_specs=[_row_spec(_BRF, 48), _degp_spec(_BRF),
                  _accp_spec(_BRF, 48), _full_spec(1, 48)],
        out_specs=_row_spec(_BRF, 40),
        out_shape=jax.ShapeDtypeStruct((N, 40), jnp.float32),
    )(h, degp, accp, b)


def kernel(x, edge_index, W1, b1, W2, b2, W3, b3):
    pad_idx = jnp.full((E_ALLOC - E,), N, jnp.int32)
    src = jnp.concatenate([edge_index[0].astype(jnp.int32), pad_idx])
    dst = jnp.concatenate([edge_index[1].astype(jnp.int32), pad_idx])
    src = src.reshape(TOT_CHUNKS + NCH_MAX, CHUNK)
    dst = dst.reshape(TOT_CHUNKS + NCH_MAX, CHUNK)
    xp = jnp.pad(x, ((0, N_PAD - N), (0, 0)))

    z16 = jnp.zeros((RPS, 16), jnp.float32)
    z32 = jnp.zeros((RPS, 32), jnp.float32)
    z48 = jnp.zeros((RPS, 48), jnp.float32)
    ones = jnp.ones((CHUNK, 16), jnp.float32)

    W3p = jnp.pad(W3, ((0, 0), (0, 8)))
    b3p = jnp.pad(b3, (0, 8))

    deg_k, agg32, agg48, dual32 = _sc_kernels()
    degp = deg_k(dst, z16, ones)

    h1, g1a, g1b = _tc1(xp, W1, degp)
    acc1a, acc1b = dual32(g1a, g1b, src, dst, z32)
    h2, g2 = _tc_mid2(h1, degp, acc1a, acc1b, b1.reshape(1, 64), W2)
    acc2 = agg32(g2, src, dst, z32)
    h3, g3 = _tc_mid(h2, degp, acc2, b2.reshape(1, 32), W3p, 32, 48)
    acc3 = agg48(g3, src, dst, z48)
    return _tc_fin(h3, degp, acc3, b3p.reshape(1, 48))
